# Initial kernel scaffold; baseline (speedup 1.0000x reference)
import functools

import jax
import jax.numpy as jnp
from jax import lax
from jax.experimental import pallas as pl
from jax.experimental.pallas import tpu as pltpu
from jax.experimental.pallas import tpu_sc as plsc

N = 10000
F = 128
E = 320000
NC, NS = 2, 16
NW = NC * NS
NPAD = 10240
K = 128
NCH = 80
EPW = NCH * K
EPAD = EPW * NW
PADNODE = N + 100
RPT = NPAD // NS
DW = 16

_MESH = plsc.VectorSubcoreMesh(core_axis_name="c", subcore_axis_name="s")


def _deg_body(dst_hbm, out_hbm, didx, ones, zbuf, acc):
    c = lax.axis_index("c")
    s = lax.axis_index("s")
    wid = s * NC + c

    def fill(i, carry):
        zbuf[i] = jnp.zeros((16,), jnp.float32)
        ones[i] = jnp.full((16,), 1.0, jnp.float32)
        return carry

    lax.fori_loop(0, K, fill, 0)

    def zero(i, carry):
        pltpu.sync_copy(zbuf, acc.at[pl.ds(s * RPT + i * K, K)])
        return carry

    lax.fori_loop(0, RPT // K, zero, 0)
    plsc.subcore_barrier()

    def chunk(i, carry):
        pltpu.sync_copy(dst_hbm.at[wid, i], didx)
        pltpu.sync_copy(ones, acc.at[didx], add=True)
        return carry

    lax.fori_loop(0, NCH, chunk, 0)
    plsc.subcore_barrier()
    pltpu.sync_copy(acc.at[pl.ds(s * RPT, RPT)],
                    out_hbm.at[c, pl.ds(s * RPT, RPT)])


_sc_degree = functools.partial(
    pl.kernel,
    out_type=jax.ShapeDtypeStruct((NC, NPAD, DW), jnp.float32),
    mesh=_MESH,
    scratch_types=[
        pltpu.VMEM((K,), jnp.int32),
        pltpu.VMEM((K, DW), jnp.float32),
        pltpu.VMEM((K, DW), jnp.float32),
        pltpu.VMEM_SHARED((NPAD, DW), jnp.float32),
    ],
)(_deg_body)


def _agg_body(hs_hbm, src_hbm, dst_hbm, out_hbm, sidx, didx, rows, zbuf, acc,
              sem):
    c = lax.axis_index("c")
    s = lax.axis_index("s")
    wid = s * NC + c

    def fill(r, carry):
        for j in range(8):
            zbuf[r, pl.ds(j * 16, 16)] = jnp.zeros((16,), jnp.float32)
        return carry

    lax.fori_loop(0, K, fill, 0)

    def zero(i, carry):
        pltpu.sync_copy(zbuf, acc.at[pl.ds(s * RPT + i * K, K)])
        return carry

    lax.fori_loop(0, RPT // K, zero, 0)
    plsc.subcore_barrier()

    pltpu.sync_copy(src_hbm.at[wid], sidx)
    pltpu.sync_copy(dst_hbm.at[wid], didx)

    def chunk(i, carry):
        pltpu.async_copy(hs_hbm.at[sidx.at[i]], rows, sem).wait()
        pltpu.sync_copy(rows, acc.at[didx.at[i]], add=True)
        return carry

    lax.fori_loop(0, NCH, chunk, 0)
    plsc.subcore_barrier()
    pltpu.sync_copy(acc.at[pl.ds(s * RPT, RPT)],
                    out_hbm.at[c, pl.ds(s * RPT, RPT)])


_sc_aggregate = functools.partial(
    pl.kernel,
    out_type=jax.ShapeDtypeStruct((NC, NPAD, F), jnp.float32),
    mesh=_MESH,
    scratch_types=[
        pltpu.VMEM((NCH, K), jnp.int32),
        pltpu.VMEM((NCH, K), jnp.int32),
        pltpu.VMEM((K, F), jnp.float32),
        pltpu.VMEM((K, F), jnp.float32),
        pltpu.VMEM_SHARED((NPAD, F), jnp.float32),
        pltpu.SemaphoreType.DMA,
    ],
)(_agg_body)


BLK = 1024


def _dis_from(degp_ref):
    deg = degp_ref[0, :, 0:1] + degp_ref[1, :, 0:1] + 1.0
    return lax.rsqrt(deg)


def _prep_body(x_ref, w_ref, degp_ref, h_ref, hs_ref):
    h = jnp.dot(x_ref[...], w_ref[...], preferred_element_type=jnp.float32)
    dis = _dis_from(degp_ref)
    h_ref[...] = h
    hs_ref[...] = h * dis


def _mid_body(p_ref, h1_ref, degp_ref, b_ref, w_ref, h2_ref, hs2_ref):
    dis = _dis_from(degp_ref)
    agg = p_ref[0] + p_ref[1]
    x2 = jnp.maximum(dis * agg + (dis * dis) * h1_ref[...] + b_ref[...], 0.0)
    h2 = jnp.dot(x2, w_ref[...], preferred_element_type=jnp.float32)
    h2_ref[...] = h2
    hs2_ref[...] = h2 * dis


def _fin_body(p_ref, h2_ref, degp_ref, b_ref, out_ref):
    dis = _dis_from(degp_ref)
    agg = p_ref[0] + p_ref[1]
    out_ref[...] = dis * agg + (dis * dis) * h2_ref[...] + b_ref[...]


def _row_spec():
    return pl.BlockSpec((BLK, F), lambda i: (i, 0))


def _part_spec():
    return pl.BlockSpec((NC, BLK, F), lambda i: (0, i, 0))


def _deg_spec():
    return pl.BlockSpec((NC, BLK, DW), lambda i: (0, i, 0))


def _w_spec():
    return pl.BlockSpec((F, F), lambda i: (0, 0))


def _b_spec():
    return pl.BlockSpec((1, F), lambda i: (0, 0))


_GRID = (NPAD // BLK,)

_tc_prep = pl.pallas_call(
    _prep_body,
    grid=_GRID,
    in_specs=[_row_spec(), _w_spec(), _deg_spec()],
    out_specs=[_row_spec(), _row_spec()],
    out_shape=[jax.ShapeDtypeStruct((NPAD, F), jnp.float32)] * 2,
)

_tc_mid = pl.pallas_call(
    _mid_body,
    grid=_GRID,
    in_specs=[_part_spec(), _row_spec(), _deg_spec(), _b_spec(), _w_spec()],
    out_specs=[_row_spec(), _row_spec()],
    out_shape=[jax.ShapeDtypeStruct((NPAD, F), jnp.float32)] * 2,
)

_tc_fin = pl.pallas_call(
    _fin_body,
    grid=_GRID,
    in_specs=[_part_spec(), _row_spec(), _deg_spec(), _b_spec()],
    out_specs=_row_spec(),
    out_shape=jax.ShapeDtypeStruct((NPAD, F), jnp.float32),
)


def kernel(node_features, edge_index, W1, b1, W2, b2):
    ei = edge_index.astype(jnp.int32)
    pad = jnp.full((EPAD - E,), PADNODE, jnp.int32)
    srcp = jnp.concatenate([ei[0], pad]).reshape(NW, NCH, K)
    dstp = jnp.concatenate([ei[1], pad]).reshape(NW, NCH, K)
    x_p = jnp.concatenate(
        [node_features, jnp.zeros((NPAD - N, F), jnp.float32)])
    b1r = b1.reshape(1, F)
    b2r = b2.reshape(1, F)

    degp = _sc_degree(dstp)
    h1, hs1 = _tc_prep(x_p, W1, degp)
    parts1 = _sc_aggregate(hs1, srcp, dstp)
    h2, hs2 = _tc_mid(parts1, h1, degp, b1r, W2)
    parts2 = _sc_aggregate(hs2, srcp, dstp)
    out = _tc_fin(parts2, h2, degp, b2r)
    return out[:N]


# speedup vs baseline: 8.4290x; 8.4290x over previous
import functools

import jax
import jax.numpy as jnp
from jax import lax
from jax.experimental import pallas as pl
from jax.experimental.pallas import tpu as pltpu
from jax.experimental.pallas import tpu_sc as plsc

N = 10000
F = 128
E = 320000
NC, NS = 2, 16
NW = NC * NS
NPAD = 10240
K = 128
NCH = 80
EPW = NCH * K
EPAD = EPW * NW
PADNODE = N + 100
RPT = NPAD // NS
DW = 16

_MESH = plsc.VectorSubcoreMesh(core_axis_name="c", subcore_axis_name="s",
                               num_cores=NC, num_subcores=NS)


def _deg_body(dst_hbm, out_hbm, didx, ones, zbuf, acc):
    c = lax.axis_index("c")
    s = lax.axis_index("s")
    wid = s * NC + c

    def fill(i, carry):
        zbuf[i] = jnp.zeros((16,), jnp.float32)
        ones[i] = jnp.full((16,), 1.0, jnp.float32)
        return carry

    lax.fori_loop(0, K, fill, 0)

    def zero(i, carry):
        pltpu.sync_copy(zbuf, acc.at[pl.ds(s * RPT + i * K, K)])
        return carry

    lax.fori_loop(0, RPT // K, zero, 0)
    plsc.subcore_barrier()

    def chunk(i, carry):
        pltpu.sync_copy(dst_hbm.at[wid, i], didx)
        pltpu.sync_copy(ones, acc.at[didx], add=True)
        return carry

    lax.fori_loop(0, NCH, chunk, 0)
    plsc.subcore_barrier()
    pltpu.sync_copy(acc.at[pl.ds(s * RPT, RPT)],
                    out_hbm.at[c, pl.ds(s * RPT, RPT)])


_sc_degree = functools.partial(
    pl.kernel,
    out_type=jax.ShapeDtypeStruct((NC, NPAD, DW), jnp.float32),
    mesh=_MESH,
    scratch_types=[
        pltpu.VMEM((K,), jnp.int32),
        pltpu.VMEM((K, DW), jnp.float32),
        pltpu.VMEM((K, DW), jnp.float32),
        pltpu.VMEM_SHARED((NPAD, DW), jnp.float32),
    ],
)(_deg_body)


def _agg_body(hs_hbm, src_hbm, dst_hbm, out_hbm, sidx, didx, rows, acc, sem):
    c = lax.axis_index("c")
    s = lax.axis_index("s")
    wid = s * NC + c

    def fill(r, carry):
        for j in range(8):
            rows[r, pl.ds(j * 16, 16)] = jnp.zeros((16,), jnp.float32)
        return carry

    lax.fori_loop(0, K, fill, 0)

    def zero(i, carry):
        pltpu.sync_copy(rows, acc.at[pl.ds(s * RPT + i * K, K)])
        return carry

    lax.fori_loop(0, RPT // K, zero, 0)
    plsc.subcore_barrier()

    def chunk(i, carry):
        pltpu.sync_copy(src_hbm.at[wid, i], sidx)
        pltpu.sync_copy(dst_hbm.at[wid, i], didx)
        pltpu.async_copy(hs_hbm.at[sidx], rows, sem).wait()
        pltpu.sync_copy(rows, acc.at[didx], add=True)
        return carry

    lax.fori_loop(0, NCH, chunk, 0)
    plsc.subcore_barrier()
    pltpu.sync_copy(acc.at[pl.ds(s * RPT, RPT)],
                    out_hbm.at[c, pl.ds(s * RPT, RPT)])


_sc_aggregate = functools.partial(
    pl.kernel,
    out_type=jax.ShapeDtypeStruct((NC, NPAD, F), jnp.float32),
    mesh=_MESH,
    scratch_types=[
        pltpu.VMEM((K,), jnp.int32),
        pltpu.VMEM((K,), jnp.int32),
        pltpu.VMEM((K, F), jnp.float32),
        pltpu.VMEM_SHARED((NPAD, F), jnp.float32),
        pltpu.SemaphoreType.DMA,
    ],
)(_agg_body)


BLK = 1024


def _dis_from(degp_ref):
    deg = degp_ref[0, :, 0:1] + degp_ref[1, :, 0:1] + 1.0
    return lax.rsqrt(deg)


def _prep_body(x_ref, w_ref, degp_ref, h_ref, hs_ref):
    h = jnp.dot(x_ref[...], w_ref[...], preferred_element_type=jnp.float32)
    dis = _dis_from(degp_ref)
    h_ref[...] = h
    hs_ref[...] = h * dis


def _mid_body(p_ref, h1_ref, degp_ref, b_ref, w_ref, h2_ref, hs2_ref):
    dis = _dis_from(degp_ref)
    agg = p_ref[0] + p_ref[1]
    x2 = jnp.maximum(dis * agg + (dis * dis) * h1_ref[...] + b_ref[...], 0.0)
    h2 = jnp.dot(x2, w_ref[...], preferred_element_type=jnp.float32)
    h2_ref[...] = h2
    hs2_ref[...] = h2 * dis


def _fin_body(p_ref, h2_ref, degp_ref, b_ref, out_ref):
    dis = _dis_from(degp_ref)
    agg = p_ref[0] + p_ref[1]
    out_ref[...] = dis * agg + (dis * dis) * h2_ref[...] + b_ref[...]


def _row_spec():
    return pl.BlockSpec((BLK, F), lambda i: (i, 0))


def _part_spec():
    return pl.BlockSpec((NC, BLK, F), lambda i: (0, i, 0))


def _deg_spec():
    return pl.BlockSpec((NC, BLK, DW), lambda i: (0, i, 0))


def _w_spec():
    return pl.BlockSpec((F, F), lambda i: (0, 0))


def _b_spec():
    return pl.BlockSpec((1, F), lambda i: (0, 0))


_GRID = (NPAD // BLK,)

_tc_prep = pl.pallas_call(
    _prep_body,
    grid=_GRID,
    in_specs=[_row_spec(), _w_spec(), _deg_spec()],
    out_specs=[_row_spec(), _row_spec()],
    out_shape=[jax.ShapeDtypeStruct((NPAD, F), jnp.float32)] * 2,
)

_tc_mid = pl.pallas_call(
    _mid_body,
    grid=_GRID,
    in_specs=[_part_spec(), _row_spec(), _deg_spec(), _b_spec(), _w_spec()],
    out_specs=[_row_spec(), _row_spec()],
    out_shape=[jax.ShapeDtypeStruct((NPAD, F), jnp.float32)] * 2,
)

_tc_fin = pl.pallas_call(
    _fin_body,
    grid=_GRID,
    in_specs=[_part_spec(), _row_spec(), _deg_spec(), _b_spec()],
    out_specs=_row_spec(),
    out_shape=jax.ShapeDtypeStruct((NPAD, F), jnp.float32),
)


def kernel(node_features, edge_index, W1, b1, W2, b2):
    ei = edge_index.astype(jnp.int32)
    pad = jnp.full((EPAD - E,), PADNODE, jnp.int32)
    srcp = jnp.concatenate([ei[0], pad]).reshape(NW, NCH, K)
    dstp = jnp.concatenate([ei[1], pad]).reshape(NW, NCH, K)
    x_p = jnp.concatenate(
        [node_features, jnp.zeros((NPAD - N, F), jnp.float32)])
    b1r = b1.reshape(1, F)
    b2r = b2.reshape(1, F)

    degp = _sc_degree(dstp)
    h1, hs1 = _tc_prep(x_p, W1, degp)
    parts1 = _sc_aggregate(hs1, srcp, dstp)
    h2, hs2 = _tc_mid(parts1, h1, degp, b1r, W2)
    parts2 = _sc_aggregate(hs2, srcp, dstp)
    out = _tc_fin(parts2, h2, degp, b2r)
    return out[:N]


# speedup vs baseline: 9.8915x; 1.1735x over previous
import functools

import jax
import jax.numpy as jnp
from jax import lax
from jax.experimental import pallas as pl
from jax.experimental.pallas import tpu as pltpu
from jax.experimental.pallas import tpu_sc as plsc

N = 10000
F = 128
E = 320000
NC, NS = 2, 16
NW = NC * NS
NPAD = 10240
K = 128
NCH = 80
EPW = NCH * K
EPAD = EPW * NW
PADNODE = N + 100
RPT = NPAD // NS
DW = 16

_MESH = plsc.VectorSubcoreMesh(core_axis_name="c", subcore_axis_name="s",
                               num_cores=NC, num_subcores=NS)


def _deg_body(dst_hbm, out_hbm, didx, ones, zbuf, acc):
    c = lax.axis_index("c")
    s = lax.axis_index("s")
    wid = s * NC + c

    def fill(i, carry):
        zbuf[i] = jnp.zeros((16,), jnp.float32)
        ones[i] = jnp.full((16,), 1.0, jnp.float32)
        return carry

    lax.fori_loop(0, K, fill, 0)

    def zero(i, carry):
        pltpu.sync_copy(zbuf, acc.at[pl.ds(s * RPT + i * K, K)])
        return carry

    lax.fori_loop(0, RPT // K, zero, 0)
    plsc.subcore_barrier()

    def chunk(i, carry):
        pltpu.sync_copy(dst_hbm.at[wid, i], didx)
        pltpu.sync_copy(ones, acc.at[didx], add=True)
        return carry

    lax.fori_loop(0, NCH, chunk, 0)
    plsc.subcore_barrier()
    pltpu.sync_copy(acc.at[pl.ds(s * RPT, RPT)],
                    out_hbm.at[c, pl.ds(s * RPT, RPT)])


_sc_degree = functools.partial(
    pl.kernel,
    out_type=jax.ShapeDtypeStruct((NC, NPAD, DW), jnp.float32),
    mesh=_MESH,
    scratch_types=[
        pltpu.VMEM((K,), jnp.int32),
        pltpu.VMEM((K, DW), jnp.float32),
        pltpu.VMEM((K, DW), jnp.float32),
        pltpu.VMEM_SHARED((NPAD, DW), jnp.float32),
    ],
)(_deg_body)


def _agg_body(hs_hbm, src_hbm, dst_hbm, out_hbm,
              sidxa, didxa, sidxb, didxb, rowsa, rowsb, acc,
              sga, sgb, ssa, ssb):
    c = lax.axis_index("c")
    s = lax.axis_index("s")
    wid = s * NC + c

    def fill(r, carry):
        for j in range(8):
            rowsa[r, pl.ds(j * 16, 16)] = jnp.zeros((16,), jnp.float32)
        return carry

    lax.fori_loop(0, K, fill, 0)

    def zero(i, carry):
        pltpu.sync_copy(rowsa, acc.at[pl.ds(s * RPT + i * K, K)])
        return carry

    lax.fori_loop(0, RPT // K, zero, 0)
    plsc.subcore_barrier()

    def load_and_gather(i, sidx_, didx_, rows_, gsem_):
        pltpu.sync_copy(src_hbm.at[wid, i], sidx_)
        pltpu.sync_copy(dst_hbm.at[wid, i], didx_)
        pltpu.async_copy(hs_hbm.at[sidx_], rows_, gsem_)

    load_and_gather(0, sidxa, didxa, rowsa, sga)
    load_and_gather(1, sidxb, didxb, rowsb, sgb)

    def body(j, carry):
        pltpu.make_async_copy(hs_hbm.at[sidxa], rowsa, sga).wait()
        pltpu.async_copy(rowsa, acc.at[didxa], ssa, add=True)
        pltpu.make_async_copy(hs_hbm.at[sidxb], rowsb, sgb).wait()
        pltpu.async_copy(rowsb, acc.at[didxb], ssb, add=True)

        @pl.when(j < NCH // 2 - 1)
        def _():
            pltpu.make_async_copy(rowsa, acc.at[didxa], ssa).wait()
            load_and_gather(2 * j + 2, sidxa, didxa, rowsa, sga)
            pltpu.make_async_copy(rowsb, acc.at[didxb], ssb).wait()
            load_and_gather(2 * j + 3, sidxb, didxb, rowsb, sgb)

        return carry

    lax.fori_loop(0, NCH // 2, body, 0)
    pltpu.make_async_copy(rowsa, acc.at[didxa], ssa).wait()
    pltpu.make_async_copy(rowsb, acc.at[didxb], ssb).wait()
    plsc.subcore_barrier()
    pltpu.sync_copy(acc.at[pl.ds(s * RPT, RPT)],
                    out_hbm.at[c, pl.ds(s * RPT, RPT)])


_sc_aggregate = functools.partial(
    pl.kernel,
    out_type=jax.ShapeDtypeStruct((NC, NPAD, F), jnp.float32),
    mesh=_MESH,
    scratch_types=[
        pltpu.VMEM((K,), jnp.int32),
        pltpu.VMEM((K,), jnp.int32),
        pltpu.VMEM((K,), jnp.int32),
        pltpu.VMEM((K,), jnp.int32),
        pltpu.VMEM((K, F), jnp.float32),
        pltpu.VMEM((K, F), jnp.float32),
        pltpu.VMEM_SHARED((NPAD, F), jnp.float32),
        pltpu.SemaphoreType.DMA,
        pltpu.SemaphoreType.DMA,
        pltpu.SemaphoreType.DMA,
        pltpu.SemaphoreType.DMA,
    ],
)(_agg_body)


BLK = 1024


def _dis_from(degp_ref):
    deg = degp_ref[0, :, 0:1] + degp_ref[1, :, 0:1] + 1.0
    return lax.rsqrt(deg)


def _prep_body(x_ref, w_ref, degp_ref, h_ref, hs_ref):
    h = jnp.dot(x_ref[...], w_ref[...], preferred_element_type=jnp.float32)
    dis = _dis_from(degp_ref)
    h_ref[...] = h
    hs_ref[...] = h * dis


def _mid_body(p_ref, h1_ref, degp_ref, b_ref, w_ref, h2_ref, hs2_ref):
    dis = _dis_from(degp_ref)
    agg = p_ref[0] + p_ref[1]
    x2 = jnp.maximum(dis * agg + (dis * dis) * h1_ref[...] + b_ref[...], 0.0)
    h2 = jnp.dot(x2, w_ref[...], preferred_element_type=jnp.float32)
    h2_ref[...] = h2
    hs2_ref[...] = h2 * dis


def _fin_body(p_ref, h2_ref, degp_ref, b_ref, out_ref):
    dis = _dis_from(degp_ref)
    agg = p_ref[0] + p_ref[1]
    out_ref[...] = dis * agg + (dis * dis) * h2_ref[...] + b_ref[...]


def _row_spec():
    return pl.BlockSpec((BLK, F), lambda i: (i, 0))


def _part_spec():
    return pl.BlockSpec((NC, BLK, F), lambda i: (0, i, 0))


def _deg_spec():
    return pl.BlockSpec((NC, BLK, DW), lambda i: (0, i, 0))


def _w_spec():
    return pl.BlockSpec((F, F), lambda i: (0, 0))


def _b_spec():
    return pl.BlockSpec((1, F), lambda i: (0, 0))


_GRID = (NPAD // BLK,)

_tc_prep = pl.pallas_call(
    _prep_body,
    grid=_GRID,
    in_specs=[_row_spec(), _w_spec(), _deg_spec()],
    out_specs=[_row_spec(), _row_spec()],
    out_shape=[jax.ShapeDtypeStruct((NPAD, F), jnp.float32)] * 2,
)

_tc_mid = pl.pallas_call(
    _mid_body,
    grid=_GRID,
    in_specs=[_part_spec(), _row_spec(), _deg_spec(), _b_spec(), _w_spec()],
    out_specs=[_row_spec(), _row_spec()],
    out_shape=[jax.ShapeDtypeStruct((NPAD, F), jnp.float32)] * 2,
)

_tc_fin = pl.pallas_call(
    _fin_body,
    grid=_GRID,
    in_specs=[_part_spec(), _row_spec(), _deg_spec(), _b_spec()],
    out_specs=_row_spec(),
    out_shape=jax.ShapeDtypeStruct((NPAD, F), jnp.float32),
)


def kernel(node_features, edge_index, W1, b1, W2, b2):
    ei = edge_index.astype(jnp.int32)
    pad = jnp.full((EPAD - E,), PADNODE, jnp.int32)
    srcp = jnp.concatenate([ei[0], pad]).reshape(NW, NCH, K)
    dstp = jnp.concatenate([ei[1], pad]).reshape(NW, NCH, K)
    x_p = jnp.concatenate(
        [node_features, jnp.zeros((NPAD - N, F), jnp.float32)])
    b1r = b1.reshape(1, F)
    b2r = b2.reshape(1, F)

    degp = _sc_degree(dstp)
    h1, hs1 = _tc_prep(x_p, W1, degp)
    parts1 = _sc_aggregate(hs1, srcp, dstp)
    h2, hs2 = _tc_mid(parts1, h1, degp, b1r, W2)
    parts2 = _sc_aggregate(hs2, srcp, dstp)
    out = _tc_fin(parts2, h2, degp, b2r)
    return out[:N]


# speedup vs baseline: 18.4893x; 1.8692x over previous
import functools

import jax
import jax.numpy as jnp
from jax import lax
from jax.experimental import pallas as pl
from jax.experimental.pallas import tpu as pltpu
from jax.experimental.pallas import tpu_sc as plsc

N = 10000
F = 128
E = 320000
NC, NS = 2, 16
NW = NC * NS
NPAD = 10240
K = 128
NCH = 80
EPW = NCH * K
EPAD = EPW * NW
PADNODE = N + 100
RPT = NPAD // NS
DW = 16
TCH = EPAD // K
N0, N1 = 48, 112

_MESH = plsc.VectorSubcoreMesh(core_axis_name="c", subcore_axis_name="s",
                               num_cores=NC, num_subcores=NS)


def _deg_body(dst_hbm, out_hbm, didx, ones, zbuf, acc):
    c = lax.axis_index("c")
    s = lax.axis_index("s")
    wid = s * NC + c

    def fill(i, carry):
        zbuf[i] = jnp.zeros((16,), jnp.float32)
        ones[i] = jnp.full((16,), 1.0, jnp.float32)
        return carry

    lax.fori_loop(0, K, fill, 0)

    def zero(i, carry):
        pltpu.sync_copy(zbuf, acc.at[pl.ds(s * RPT + i * K, K)])
        return carry

    lax.fori_loop(0, RPT // K, zero, 0)
    plsc.subcore_barrier()

    def chunk(i, carry):
        pltpu.sync_copy(dst_hbm.at[wid * NCH + i], didx)
        pltpu.sync_copy(ones, acc.at[didx], add=True)
        return carry

    lax.fori_loop(0, NCH, chunk, 0)
    plsc.subcore_barrier()
    pltpu.sync_copy(acc.at[pl.ds(s * RPT, RPT)],
                    out_hbm.at[c, pl.ds(s * RPT, RPT)])


_sc_degree = functools.partial(
    pl.kernel,
    out_type=jax.ShapeDtypeStruct((NC, NPAD, DW), jnp.float32),
    mesh=_MESH,
    scratch_types=[
        pltpu.VMEM((K,), jnp.int32),
        pltpu.VMEM((K, DW), jnp.float32),
        pltpu.VMEM((K, DW), jnp.float32),
        pltpu.VMEM_SHARED((NPAD, DW), jnp.float32),
    ],
)(_deg_body)


def _agg_body(hs_hbm, src_hbm, dst_hbm, out_hbm,
              sidxa, didxa, sidxb, didxb, rowsa, rowsb, acc,
              sga, sgb, ssa, ssb):
    c = lax.axis_index("c")
    s = lax.axis_index("s")
    wid = s * NC + c

    def fill(r, carry):
        for j in range(8):
            rowsa[r, pl.ds(j * 16, 16)] = jnp.zeros((16,), jnp.float32)
        return carry

    lax.fori_loop(0, K, fill, 0)

    def zero(i, carry):
        pltpu.sync_copy(rowsa, acc.at[pl.ds(s * RPT + i * K, K)])
        return carry

    lax.fori_loop(0, RPT // K, zero, 0)
    plsc.subcore_barrier()

    start = jnp.where(c == 0, s * N0, NS * N0 + s * N1)
    half = jnp.where(c == 0, N0 // 2, N1 // 2)

    def load_and_gather(i, sidx_, didx_, rows_, gsem_):
        pltpu.sync_copy(src_hbm.at[start + i], sidx_)
        pltpu.sync_copy(dst_hbm.at[start + i], didx_)
        pltpu.async_copy(hs_hbm.at[sidx_], rows_, gsem_)

    load_and_gather(0, sidxa, didxa, rowsa, sga)
    load_and_gather(1, sidxb, didxb, rowsb, sgb)

    def body(j, carry):
        pltpu.make_async_copy(hs_hbm.at[sidxa], rowsa, sga).wait()
        pltpu.async_copy(rowsa, acc.at[didxa], ssa, add=True)
        pltpu.make_async_copy(hs_hbm.at[sidxb], rowsb, sgb).wait()
        pltpu.async_copy(rowsb, acc.at[didxb], ssb, add=True)

        @pl.when(j < half - 1)
        def _():
            pltpu.make_async_copy(rowsa, acc.at[didxa], ssa).wait()
            load_and_gather(2 * j + 2, sidxa, didxa, rowsa, sga)
            pltpu.make_async_copy(rowsb, acc.at[didxb], ssb).wait()
            load_and_gather(2 * j + 3, sidxb, didxb, rowsb, sgb)

        return carry

    lax.fori_loop(0, half, body, 0)
    pltpu.make_async_copy(rowsa, acc.at[didxa], ssa).wait()
    pltpu.make_async_copy(rowsb, acc.at[didxb], ssb).wait()
    plsc.subcore_barrier()
    pltpu.sync_copy(acc.at[pl.ds(s * RPT, RPT)],
                    out_hbm.at[c, pl.ds(s * RPT, RPT)])


_sc_aggregate = functools.partial(
    pl.kernel,
    out_type=jax.ShapeDtypeStruct((NC, NPAD, F), jnp.float32),
    mesh=_MESH,
    scratch_types=[
        pltpu.VMEM((K,), jnp.int32),
        pltpu.VMEM((K,), jnp.int32),
        pltpu.VMEM((K,), jnp.int32),
        pltpu.VMEM((K,), jnp.int32),
        pltpu.VMEM((K, F), jnp.float32),
        pltpu.VMEM((K, F), jnp.float32),
        pltpu.VMEM_SHARED((NPAD, F), jnp.float32),
        pltpu.SemaphoreType.DMA,
        pltpu.SemaphoreType.DMA,
        pltpu.SemaphoreType.DMA,
        pltpu.SemaphoreType.DMA,
    ],
)(_agg_body)


BLK = 1024


def _dis_from(degp_ref):
    deg = degp_ref[0, :, 0:1] + degp_ref[1, :, 0:1] + 1.0
    return lax.rsqrt(deg)


def _prep_body(x_ref, w_ref, degp_ref, h_ref, hs_ref):
    h = jnp.dot(x_ref[...], w_ref[...], preferred_element_type=jnp.float32)
    dis = _dis_from(degp_ref)
    h_ref[...] = h
    hs_ref[...] = h * dis


def _mid_body(p_ref, h1_ref, degp_ref, b_ref, w_ref, h2_ref, hs2_ref):
    dis = _dis_from(degp_ref)
    agg = p_ref[0] + p_ref[1]
    x2 = jnp.maximum(dis * agg + (dis * dis) * h1_ref[...] + b_ref[...], 0.0)
    h2 = jnp.dot(x2, w_ref[...], preferred_element_type=jnp.float32)
    h2_ref[...] = h2
    hs2_ref[...] = h2 * dis


def _fin_body(p_ref, h2_ref, degp_ref, b_ref, out_ref):
    dis = _dis_from(degp_ref)
    agg = p_ref[0] + p_ref[1]
    out_ref[...] = dis * agg + (dis * dis) * h2_ref[...] + b_ref[...]


def _row_spec():
    return pl.BlockSpec((BLK, F), lambda i: (i, 0))


def _part_spec():
    return pl.BlockSpec((NC, BLK, F), lambda i: (0, i, 0))


def _deg_spec():
    return pl.BlockSpec((NC, BLK, DW), lambda i: (0, i, 0))


def _w_spec():
    return pl.BlockSpec((F, F), lambda i: (0, 0))


def _b_spec():
    return pl.BlockSpec((1, F), lambda i: (0, 0))


_GRID = (NPAD // BLK,)

_tc_prep = pl.pallas_call(
    _prep_body,
    grid=_GRID,
    in_specs=[_row_spec(), _w_spec(), _deg_spec()],
    out_specs=[_row_spec(), _row_spec()],
    out_shape=[jax.ShapeDtypeStruct((NPAD, F), jnp.float32)] * 2,
)

_tc_mid = pl.pallas_call(
    _mid_body,
    grid=_GRID,
    in_specs=[_part_spec(), _row_spec(), _deg_spec(), _b_spec(), _w_spec()],
    out_specs=[_row_spec(), _row_spec()],
    out_shape=[jax.ShapeDtypeStruct((NPAD, F), jnp.float32)] * 2,
)

_tc_fin = pl.pallas_call(
    _fin_body,
    grid=_GRID,
    in_specs=[_part_spec(), _row_spec(), _deg_spec(), _b_spec()],
    out_specs=_row_spec(),
    out_shape=jax.ShapeDtypeStruct((NPAD, F), jnp.float32),
)


def kernel(node_features, edge_index, W1, b1, W2, b2):
    ei = edge_index.astype(jnp.int32)
    pad = N + jnp.arange(EPAD - E, dtype=jnp.int32) % (NPAD - N)
    srcp = jnp.concatenate([ei[0], pad]).reshape(TCH, K)
    dstp = jnp.concatenate([ei[1], pad]).reshape(TCH, K)
    x_p = jnp.concatenate(
        [node_features, jnp.zeros((NPAD - N, F), jnp.float32)])
    b1r = b1.reshape(1, F)
    b2r = b2.reshape(1, F)

    degp = _sc_degree(dstp)
    h1, hs1 = _tc_prep(x_p, W1, degp)
    parts1 = _sc_aggregate(hs1, srcp, dstp)
    h2, hs2 = _tc_mid(parts1, h1, degp, b1r, W2)
    parts2 = _sc_aggregate(hs2, srcp, dstp)
    out = _tc_fin(parts2, h2, degp, b2r)
    return out[:N]
